# baseline (speedup 1.0000x reference)
import jax
import jax.numpy as jnp
from jax import lax
from jax.experimental import pallas as pl
from jax.experimental.pallas import tpu as pltpu
from jax.experimental.pallas import tpu_sc as plsc

SEQ = 200
HID = 64
BATCH = 4096

_info = plsc.get_sparse_core_info()
NC, NS = _info.num_cores, _info.num_subcores
NW = NC * NS
BT = BATCH // NW
NBUF = 4
LAG = 3
HT = HID // 8
NBT = BATCH // 128
TILE = 8 * 128


def _emb_body(xt_hbm, bpe_hbm, pos_hbm, out_hbm, xt_v, pos_v, *bufs):
    rows = bufs[:NBUF]
    tbuf = bufs[NBUF:2 * NBUF]
    gsem = bufs[2 * NBUF:3 * NBUF]
    osem = bufs[3 * NBUF:4 * NBUF]

    wid = lax.axis_index("s") * NC + lax.axis_index("c")
    c0 = wid * BT

    pltpu.sync_copy(xt_hbm.at[:, wid], xt_v)
    pltpu.sync_copy(pos_hbm.at[pl.ds(0, SEQ)], pos_v)

    biota = lax.iota(jnp.int32, 16)

    def start_gather(slot, buf):
        pltpu.async_copy(
            bpe_hbm.at[xt_v.at[slot // 8, slot % 8]], rows[buf], gsem[buf])

    for k in range(LAG):
        start_gather(k, k % NBUF)

    def outer(i, _):
        for b in range(NBUF):
            s = i * NBUF + b
            bp = (b + LAG) % NBUF

            @pl.when(s + LAG < SEQ)
            def _():
                start_gather(s + LAG, bp)

            pltpu.make_async_copy(
                bpe_hbm.at[xt_v.at[s]], rows[b], gsem[b]).wait()

            @pl.when(s >= NBUF)
            def _():
                for ht in range(HT):
                    pltpu.make_async_copy(
                        tbuf[b].at[pl.ds(0, 8), :128],
                        out_hbm.at[0, ht, 0], osem[b]).wait()

            pos_g = [pos_v[s, pl.ds(g * 16, 16)] for g in range(HID // 16)]
            row_g = [biota + g * 16 for g in range(HID // 16)]

            @plsc.parallel_loop(0, BT, unroll=32)
            def _(r):
                rvec = jnp.broadcast_to(jnp.int32(r), (16,))
                for g in range(HID // 16):
                    v = rows[b][r, pl.ds(g * 16, 16)] + pos_g[g]
                    plsc.store_scatter(tbuf[b], [row_g[g], rvec], v)

            for ht in range(HT):
                pltpu.async_copy(
                    tbuf[b].at[pl.ds(ht * 8, 8), :128],
                    out_hbm.at[s, ht, wid], osem[b])
        return 0

    lax.fori_loop(0, SEQ // NBUF, outer, 0)

    for b in range(NBUF):
        for ht in range(HT):
            pltpu.make_async_copy(
                tbuf[b].at[pl.ds(0, 8), :128],
                out_hbm.at[0, ht, 0], osem[b]).wait()


@jax.jit
def _emb(x, bpe_table, pos_table):
    mesh = plsc.VectorSubcoreMesh(core_axis_name="c", subcore_axis_name="s")
    f = pl.kernel(
        _emb_body,
        out_type=jax.ShapeDtypeStruct((SEQ, HT, NBT, 8, 128), jnp.float32),
        mesh=mesh,
        scratch_types=(
            [pltpu.VMEM((SEQ // 8, 8, 128), jnp.int32),
             pltpu.VMEM((SEQ, HID), jnp.float32)]
            + [pltpu.VMEM((BT, HID), jnp.float32) for _ in range(NBUF)]
            + [pltpu.VMEM((HID, BT + 1), jnp.float32) for _ in range(NBUF)]
            + [pltpu.SemaphoreType.DMA for _ in range(2 * NBUF)]
        ),
        compiler_params=pltpu.CompilerParams(
            use_tc_tiling_on_sc=False, needs_layout_passes=False),
    )
    x_tiled = x.T.reshape(SEQ // 8, 8, NBT, 128).transpose(0, 2, 1, 3)
    phys = f(x_tiled, bpe_table, pos_table)
    return phys.transpose(2, 4, 0, 1, 3).reshape(BATCH, SEQ, HID)


def kernel(x, bpe_table, pos_table):
    return _emb(x, bpe_table, pos_table)


try:
    import sys as _sys
    if any("validate" in _a for _a in _sys.argv):
        _k = jax.jit(kernel).lower(
            jax.ShapeDtypeStruct((4096, 200), jnp.int32),
            jax.ShapeDtypeStruct((100000, 64), jnp.float32),
            jax.ShapeDtypeStruct((512, 64), jnp.float32),
        ).compile().as_text()
        for _line in _k.splitlines():
            _t = _line.strip()
            if _t.startswith("%") or _t.startswith("ROOT"):
                print("PROBE-K:", _t.split(" metadata=")[0].split(
                    ", backend_config=")[0][:200])
except Exception as _e:
    print("PROBE-ERR:", _e)


# speedup vs baseline: 1.3057x; 1.3057x over previous
import jax
import jax.numpy as jnp
from jax import lax
from jax.experimental import pallas as pl
from jax.experimental.pallas import tpu as pltpu
from jax.experimental.pallas import tpu_sc as plsc

SEQ = 200
HID = 64
BATCH = 4096

_info = plsc.get_sparse_core_info()
NC, NS = _info.num_cores, _info.num_subcores
NW = NC * NS
BT = BATCH // NW
NBUF = 4
LAG = 3
HT = HID // 8
NBT = BATCH // 128
TILE = 8 * 128


def _emb_body(xt_hbm, bpe_hbm, pos_hbm, out_hbm, xt_v, pos_v, *bufs):
    rows = bufs[:NBUF]
    tbuf = bufs[NBUF:2 * NBUF]
    gsem = bufs[2 * NBUF:3 * NBUF]
    osem = bufs[3 * NBUF:4 * NBUF]

    wid = lax.axis_index("s") * NC + lax.axis_index("c")
    c0 = wid * BT

    pltpu.sync_copy(xt_hbm.at[:, wid], xt_v)
    pltpu.sync_copy(pos_hbm.at[pl.ds(0, SEQ)], pos_v)

    biota = lax.iota(jnp.int32, 16)

    def start_gather(slot, buf):
        pltpu.async_copy(
            bpe_hbm.at[xt_v.at[slot // 8, slot % 8]], rows[buf], gsem[buf])

    for k in range(LAG):
        start_gather(k, k % NBUF)

    def outer(i, _):
        for b in range(NBUF):
            s = i * NBUF + b
            bp = (b + LAG) % NBUF

            @pl.when(s + LAG < SEQ)
            def _():
                start_gather(s + LAG, bp)

            pltpu.make_async_copy(
                bpe_hbm.at[xt_v.at[s]], rows[b], gsem[b]).wait()

            @pl.when(s >= NBUF)
            def _():
                for ht in range(HT):
                    pltpu.make_async_copy(
                        tbuf[b].at[pl.ds(0, 8), :128],
                        out_hbm.at[0, ht, 0], osem[b]).wait()

            pos_g = [pos_v[s, pl.ds(g * 16, 16)] for g in range(HID // 16)]
            row_g = [biota + g * 16 for g in range(HID // 16)]

            @plsc.parallel_loop(0, BT, unroll=16)
            def _(r):
                rvec = jnp.broadcast_to(jnp.int32(r), (16,))
                for g in range(HID // 16):
                    v = rows[b][r, pl.ds(g * 16, 16)] + pos_g[g]
                    plsc.store_scatter(tbuf[b], [row_g[g], rvec], v)

            for ht in range(HT):
                pltpu.async_copy(
                    tbuf[b].at[pl.ds(ht * 8, 8), :128],
                    out_hbm.at[s, ht, wid], osem[b])
        return 0

    lax.fori_loop(0, SEQ // NBUF, outer, 0)

    for b in range(NBUF):
        for ht in range(HT):
            pltpu.make_async_copy(
                tbuf[b].at[pl.ds(0, 8), :128],
                out_hbm.at[0, ht, 0], osem[b]).wait()


@jax.jit
def _emb(x, bpe_table, pos_table):
    mesh = plsc.VectorSubcoreMesh(core_axis_name="c", subcore_axis_name="s")
    f = pl.kernel(
        _emb_body,
        out_type=jax.ShapeDtypeStruct((SEQ, HT, NBT, 8, 128), jnp.float32),
        mesh=mesh,
        scratch_types=(
            [pltpu.VMEM((SEQ // 8, 8, 128), jnp.int32),
             pltpu.VMEM((SEQ, HID), jnp.float32)]
            + [pltpu.VMEM((BT, HID), jnp.float32) for _ in range(NBUF)]
            + [pltpu.VMEM((HID, BT + 1), jnp.float32) for _ in range(NBUF)]
            + [pltpu.SemaphoreType.DMA for _ in range(2 * NBUF)]
        ),
        compiler_params=pltpu.CompilerParams(
            use_tc_tiling_on_sc=False, needs_layout_passes=False),
    )
    x_tiled = x.T.reshape(SEQ // 8, 8, NBT, 128).transpose(0, 2, 1, 3)
    phys = f(x_tiled, bpe_table, pos_table)
    return phys.transpose(2, 4, 0, 1, 3).reshape(BATCH, SEQ, HID)


def kernel(x, bpe_table, pos_table):
    return _emb(x, bpe_table, pos_table)


# speedup vs baseline: 1.4767x; 1.1309x over previous
import jax
import jax.numpy as jnp
from jax import lax
from jax.experimental import pallas as pl
from jax.experimental.pallas import tpu as pltpu
from jax.experimental.pallas import tpu_sc as plsc

SEQ = 200
HID = 64
BATCH = 4096

_info = plsc.get_sparse_core_info()
NC, NS = _info.num_cores, _info.num_subcores
NW = NC * NS
BT = BATCH // NW
NBUF = 4
LAG = 3
HT = HID // 8
NBT = BATCH // 128
TILE = 8 * 128


def _emb_body(xt_hbm, bpe_hbm, pos_hbm, out_hbm, xt_v, pos_v, *bufs):
    rows = bufs[:NBUF]
    tbuf = bufs[NBUF:2 * NBUF]
    gsem = bufs[2 * NBUF:3 * NBUF]
    osem = bufs[3 * NBUF:4 * NBUF]

    wid = lax.axis_index("s") * NC + lax.axis_index("c")
    c0 = wid * BT

    pltpu.sync_copy(xt_hbm.at[:, wid], xt_v)
    pltpu.sync_copy(pos_hbm.at[pl.ds(0, SEQ)], pos_v)

    biota = lax.iota(jnp.int32, 16)

    def start_gather(slot, buf):
        pltpu.async_copy(
            bpe_hbm.at[xt_v.at[slot // 8, slot % 8]], rows[buf], gsem[buf])

    for k in range(LAG):
        start_gather(k, k % NBUF)

    def outer(i, _):
        for b in range(NBUF):
            s = i * NBUF + b
            bp = (b + LAG) % NBUF

            @pl.when(s + LAG < SEQ)
            def _():
                start_gather(s + LAG, bp)

            pltpu.make_async_copy(
                bpe_hbm.at[xt_v.at[s]], rows[b], gsem[b]).wait()

            @pl.when(s >= NBUF)
            def _():
                for ht in range(HT):
                    pltpu.make_async_copy(
                        tbuf[b].at[pl.ds(0, 8), :128],
                        out_hbm.at[0, ht, 0], osem[b]).wait()

            pos_g = [pos_v[s, pl.ds(g * 16, 16)] for g in range(HID // 16)]
            row_g = [biota + g * 16 for g in range(HID // 16)]

            @plsc.parallel_loop(0, BT, unroll=12)
            def _(r):
                rvec = jnp.broadcast_to(jnp.int32(r), (16,))
                for g in range(HID // 16):
                    v = rows[b][r, pl.ds(g * 16, 16)] + pos_g[g]
                    plsc.store_scatter(tbuf[b], [row_g[g], rvec], v)

            for ht in range(HT):
                pltpu.async_copy(
                    tbuf[b].at[pl.ds(ht * 8, 8), :128],
                    out_hbm.at[s, ht, wid], osem[b])
        return 0

    lax.fori_loop(0, SEQ // NBUF, outer, 0)

    for b in range(NBUF):
        for ht in range(HT):
            pltpu.make_async_copy(
                tbuf[b].at[pl.ds(0, 8), :128],
                out_hbm.at[0, ht, 0], osem[b]).wait()


@jax.jit
def _emb(x, bpe_table, pos_table):
    mesh = plsc.VectorSubcoreMesh(core_axis_name="c", subcore_axis_name="s")
    f = pl.kernel(
        _emb_body,
        out_type=jax.ShapeDtypeStruct((SEQ, HT, NBT, 8, 128), jnp.float32),
        mesh=mesh,
        scratch_types=(
            [pltpu.VMEM((SEQ // 8, 8, 128), jnp.int32),
             pltpu.VMEM((SEQ, HID), jnp.float32)]
            + [pltpu.VMEM((BT, HID), jnp.float32) for _ in range(NBUF)]
            + [pltpu.VMEM((HID, BT + 1), jnp.float32) for _ in range(NBUF)]
            + [pltpu.SemaphoreType.DMA for _ in range(2 * NBUF)]
        ),
        compiler_params=pltpu.CompilerParams(
            use_tc_tiling_on_sc=False, needs_layout_passes=False),
    )
    x_tiled = x.T.reshape(SEQ // 8, 8, NBT, 128).transpose(0, 2, 1, 3)
    phys = f(x_tiled, bpe_table, pos_table)
    return phys.transpose(2, 4, 0, 1, 3).reshape(BATCH, SEQ, HID)


def kernel(x, bpe_table, pos_table):
    return _emb(x, bpe_table, pos_table)


# speedup vs baseline: 1.4778x; 1.0008x over previous
import jax
import jax.numpy as jnp
from jax import lax
from jax.experimental import pallas as pl
from jax.experimental.pallas import tpu as pltpu
from jax.experimental.pallas import tpu_sc as plsc

SEQ = 200
HID = 64
BATCH = 4096

_info = plsc.get_sparse_core_info()
NC, NS = _info.num_cores, _info.num_subcores
NW = NC * NS
BT = BATCH // NW
NBUF = 4
LAG = 3
HT = HID // 8
NBT = BATCH // 128
TILE = 8 * 128


def _emb_body(xt_hbm, bpe_hbm, pos_hbm, out_hbm, xt_v, pos_v, *bufs):
    rows = bufs[:NBUF]
    tbuf = bufs[NBUF:2 * NBUF]
    gsem = bufs[2 * NBUF:3 * NBUF]
    osem = bufs[3 * NBUF:4 * NBUF]

    wid = lax.axis_index("s") * NC + lax.axis_index("c")
    c0 = wid * BT

    pltpu.sync_copy(xt_hbm.at[:, wid], xt_v)
    pltpu.sync_copy(pos_hbm.at[pl.ds(0, SEQ)], pos_v)

    biota = lax.iota(jnp.int32, 16)

    def start_gather(slot, buf):
        pltpu.async_copy(
            bpe_hbm.at[xt_v.at[slot // 8, slot % 8]], rows[buf], gsem[buf])

    for k in range(LAG):
        start_gather(k, k % NBUF)

    def outer(i, _):
        for b in range(NBUF):
            s = i * NBUF + b
            bp = (b + LAG) % NBUF

            @pl.when(s + LAG < SEQ)
            def _():
                start_gather(s + LAG, bp)

            pltpu.make_async_copy(
                bpe_hbm.at[xt_v.at[s]], rows[b], gsem[b]).wait()

            @pl.when(s >= NBUF)
            def _():
                for ht in range(HT):
                    pltpu.make_async_copy(
                        tbuf[b].at[pl.ds(0, 8), :128],
                        out_hbm.at[0, ht, 0], osem[b]).wait()

            pos_g = [pos_v[s, pl.ds(g * 16, 16)] for g in range(HID // 16)]
            row_g = [biota + g * 16 for g in range(HID // 16)]

            @plsc.parallel_loop(0, BT, unroll=10)
            def _(r):
                rvec = jnp.broadcast_to(jnp.int32(r), (16,))
                for g in range(HID // 16):
                    v = rows[b][r, pl.ds(g * 16, 16)] + pos_g[g]
                    plsc.store_scatter(tbuf[b], [row_g[g], rvec], v)

            for ht in range(HT):
                pltpu.async_copy(
                    tbuf[b].at[pl.ds(ht * 8, 8), :128],
                    out_hbm.at[s, ht, wid], osem[b])
        return 0

    lax.fori_loop(0, SEQ // NBUF, outer, 0)

    for b in range(NBUF):
        for ht in range(HT):
            pltpu.make_async_copy(
                tbuf[b].at[pl.ds(0, 8), :128],
                out_hbm.at[0, ht, 0], osem[b]).wait()


@jax.jit
def _emb(x, bpe_table, pos_table):
    mesh = plsc.VectorSubcoreMesh(core_axis_name="c", subcore_axis_name="s")
    f = pl.kernel(
        _emb_body,
        out_type=jax.ShapeDtypeStruct((SEQ, HT, NBT, 8, 128), jnp.float32),
        mesh=mesh,
        scratch_types=(
            [pltpu.VMEM((SEQ // 8, 8, 128), jnp.int32),
             pltpu.VMEM((SEQ, HID), jnp.float32)]
            + [pltpu.VMEM((BT, HID), jnp.float32) for _ in range(NBUF)]
            + [pltpu.VMEM((HID, BT + 1), jnp.float32) for _ in range(NBUF)]
            + [pltpu.SemaphoreType.DMA for _ in range(2 * NBUF)]
        ),
        compiler_params=pltpu.CompilerParams(
            use_tc_tiling_on_sc=False, needs_layout_passes=False),
    )
    x_tiled = x.T.reshape(SEQ // 8, 8, NBT, 128).transpose(0, 2, 1, 3)
    phys = f(x_tiled, bpe_table, pos_table)
    return phys.transpose(2, 4, 0, 1, 3).reshape(BATCH, SEQ, HID)


def kernel(x, bpe_table, pos_table):
    return _emb(x, bpe_table, pos_table)


# speedup vs baseline: 1.4800x; 1.0015x over previous
import jax
import jax.numpy as jnp
from jax import lax
from jax.experimental import pallas as pl
from jax.experimental.pallas import tpu as pltpu
from jax.experimental.pallas import tpu_sc as plsc

SEQ = 200
HID = 64
BATCH = 4096

_info = plsc.get_sparse_core_info()
NC, NS = _info.num_cores, _info.num_subcores
NW = NC * NS
BT = BATCH // NW
NBUF = 4
LAG = 3
HT = HID // 8
NBT = BATCH // 128
TILE = 8 * 128


def _emb_body(xt_hbm, bpe_hbm, pos_hbm, out_hbm, xt_v, pos_v, *bufs):
    rows = bufs[:NBUF]
    tbuf = bufs[NBUF:2 * NBUF]
    gsem = bufs[2 * NBUF:3 * NBUF]
    osem = bufs[3 * NBUF:4 * NBUF]

    wid = lax.axis_index("s") * NC + lax.axis_index("c")
    c0 = wid * BT

    pltpu.sync_copy(xt_hbm.at[:, wid], xt_v)
    pltpu.sync_copy(pos_hbm.at[pl.ds(0, SEQ)], pos_v)

    biota = lax.iota(jnp.int32, 16)

    def start_gather(slot, buf):
        pltpu.async_copy(
            bpe_hbm.at[xt_v.at[slot // 8, slot % 8]], rows[buf], gsem[buf])

    for k in range(LAG):
        start_gather(k, k % NBUF)

    def outer(i, _):
        for b in range(NBUF):
            s = i * NBUF + b
            bp = (b + LAG) % NBUF

            @pl.when(s + LAG < SEQ)
            def _():
                start_gather(s + LAG, bp)

            pltpu.make_async_copy(
                bpe_hbm.at[xt_v.at[s]], rows[b], gsem[b]).wait()

            @pl.when(s >= NBUF)
            def _():
                for ht in range(HT):
                    pltpu.make_async_copy(
                        tbuf[b].at[pl.ds(0, 8), :128],
                        out_hbm.at[0, ht, 0], osem[b]).wait()

            pos_g = [pos_v[s, pl.ds(g * 16, 16)] for g in range(HID // 16)]
            row_g = [biota + g * 16 for g in range(HID // 16)]

            @plsc.parallel_loop(0, BT, unroll=12)
            def _(r):
                rvec = jnp.broadcast_to(jnp.int32(r), (16,))
                for g in range(HID // 16):
                    v = rows[b][r, pl.ds(g * 16, 16)] + pos_g[g]
                    plsc.store_scatter(tbuf[b], [row_g[g], rvec], v)

            for ht in range(HT):
                pltpu.async_copy(
                    tbuf[b].at[pl.ds(ht * 8, 8), :128],
                    out_hbm.at[s, ht, wid], osem[b])
        return 0

    lax.fori_loop(0, SEQ // NBUF, outer, 0)

    for b in range(NBUF):
        for ht in range(HT):
            pltpu.make_async_copy(
                tbuf[b].at[pl.ds(0, 8), :128],
                out_hbm.at[0, ht, 0], osem[b]).wait()


@jax.jit
def _emb(x, bpe_table, pos_table):
    mesh = plsc.VectorSubcoreMesh(core_axis_name="c", subcore_axis_name="s")
    f = pl.kernel(
        _emb_body,
        out_type=jax.ShapeDtypeStruct((SEQ, HT, NBT, 8, 128), jnp.float32),
        mesh=mesh,
        scratch_types=(
            [pltpu.VMEM((SEQ // 8, 8, 128), jnp.int32),
             pltpu.VMEM((SEQ, HID), jnp.float32)]
            + [pltpu.VMEM((BT, HID), jnp.float32) for _ in range(NBUF)]
            + [pltpu.VMEM((HID, BT + 1), jnp.float32) for _ in range(NBUF)]
            + [pltpu.SemaphoreType.DMA for _ in range(2 * NBUF)]
        ),
        compiler_params=pltpu.CompilerParams(
            use_tc_tiling_on_sc=False, needs_layout_passes=False),
    )
    x_tiled = x.T.reshape(SEQ // 8, 8, NBT, 128).transpose(0, 2, 1, 3)
    phys = f(x_tiled, bpe_table, pos_table)
    return phys.transpose(2, 4, 0, 1, 3).reshape(BATCH, SEQ, HID)


def kernel(x, bpe_table, pos_table):
    return _emb(x, bpe_table, pos_table)
